# baseline (speedup 1.0000x reference)
import jax
import jax.numpy as jnp
from jax import lax
from jax.experimental import pallas as pl
from jax.experimental.pallas import tpu as pltpu
from jax.experimental.pallas import tpu_sc as plsc

B, K, H, W = 1024, 3, 8, 8
HW = H * W
D = 128
NC, NS, L = 2, 16, 16
NW = NC * NS
BPW = B // NW
KHW = K * HW
NSTR = 2
SPI = KHW // NSTR
CH = HW // L


def _sc_body(inp_hbm, table_hbm, out_hbm,
             idx_v, acc0, acc1, accT0, accT1,
             gsem0, gsem1, osem0, osem1):
    wid = lax.axis_index("s") * NC + lax.axis_index("c")
    b0 = wid * BPW
    pltpu.sync_copy(inp_hbm.at[pl.ds(b0 * NSTR, BPW * NSTR)], idx_v)

    lanes = lax.iota(jnp.int32, L)
    row_vecs = [[k * HW + c * L + lanes for c in range(CH)] for k in range(K)]

    def fire_gathers(j, acc, gsem):
        for s in range(NSTR):
            pltpu.async_copy(table_hbm.at[idx_v.at[j * NSTR + s]],
                             acc.at[pl.ds(s * SPI, SPI)], gsem)

    def wait_gathers(acc, gsem):
        for s in range(NSTR):
            pltpu.make_async_copy(table_hbm.at[idx_v.at[s]],
                                  acc.at[pl.ds(s * SPI, SPI)], gsem).wait()

    def transpose(acc, accT):
        def per_d(d, c2):
            col = jnp.full((L,), d, dtype=jnp.int32)
            for c in range(CH):
                v = plsc.load_gather(acc, [row_vecs[0][c], col])
                v = v + plsc.load_gather(acc, [row_vecs[1][c], col])
                v = v + plsc.load_gather(acc, [row_vecs[2][c], col])
                accT[d, pl.ds(c * L, L)] = v
            return c2
        lax.fori_loop(0, D, per_d, 0, unroll=2)

    bufs = ((acc0, accT0, gsem0, osem0), (acc1, accT1, gsem1, osem1))

    fire_gathers(0, acc0, gsem0)
    fire_gathers(1, acc1, gsem1)

    def pipe(t, c2):
        for p, (acc, accT, gsem, osem) in enumerate(bufs):
            j = t * 2 + p
            wait_gathers(acc, gsem)

            @pl.when(j >= 2)
            def _():
                pltpu.make_async_copy(accT, out_hbm.at[pl.ds(0, D)],
                                      osem).wait()

            transpose(acc, accT)

            @pl.when(j + 2 < BPW)
            def _():
                fire_gathers(j + 2, acc, gsem)

            pltpu.async_copy(accT, out_hbm.at[pl.ds((b0 + j) * D, D)], osem)
        return c2

    lax.fori_loop(0, BPW // 2, pipe, 0)
    pltpu.make_async_copy(accT0, out_hbm.at[pl.ds(0, D)], osem0).wait()
    pltpu.make_async_copy(accT1, out_hbm.at[pl.ds(0, D)], osem1).wait()


def kernel(inputs, table):
    inp2 = inputs.reshape(B * NSTR, SPI)
    mesh = plsc.VectorSubcoreMesh(
        core_axis_name="c", subcore_axis_name="s",
        num_cores=NC, num_subcores=NS,
    )
    out = pl.kernel(
        _sc_body,
        out_type=jax.ShapeDtypeStruct((B * D, HW), jnp.float32),
        mesh=mesh,
        scratch_types=[
            pltpu.VMEM((BPW * NSTR, SPI), jnp.int32),
            pltpu.VMEM((KHW, D), jnp.float32),
            pltpu.VMEM((KHW, D), jnp.float32),
            pltpu.VMEM((D, HW), jnp.float32),
            pltpu.VMEM((D, HW), jnp.float32),
            pltpu.SemaphoreType.DMA,
            pltpu.SemaphoreType.DMA,
            pltpu.SemaphoreType.DMA,
            pltpu.SemaphoreType.DMA,
        ],
        compiler_params=pltpu.CompilerParams(needs_layout_passes=False),
    )(inp2, table)
    return out.reshape(B, D, H, W)


# speedup vs baseline: 2.5049x; 2.5049x over previous
import jax
import jax.numpy as jnp
from jax import lax
from jax.experimental import pallas as pl
from jax.experimental.pallas import tpu as pltpu
from jax.experimental.pallas import tpu_sc as plsc

B, K, H, W = 1024, 3, 8, 8
HW = H * W
D = 128
NC, NS, L = 2, 16, 16
NW = NC * NS
BPW = B // NW
CD = D // L


def _sc_body(inp_hbm, table_hbm, out_hbm,
             idx_v, acc0, acc1, accT0, accT1,
             gsem0, gsem1, osem0, osem1):
    wid = lax.axis_index("s") * NC + lax.axis_index("c")
    b0 = wid * BPW
    pltpu.sync_copy(inp_hbm.at[pl.ds(b0 * K, BPW * K)], idx_v)

    lanes = lax.iota(jnp.int32, L)
    drow_vecs = [c * L + lanes for c in range(CD)]
    zeros16 = jnp.zeros((L,), jnp.float32)

    def zero_acc(acc):
        @plsc.parallel_loop(0, HW, 1, unroll=2)
        def _(r):
            for c in range(CD):
                acc[r, pl.ds(c * L, L)] = zeros16

    def fire_gathers(j, acc, gsem):
        for k in range(K):
            pltpu.async_copy(table_hbm.at[idx_v.at[j * K + k]], acc, gsem,
                             add=True)

    def wait_gathers(acc, gsem):
        for k in range(K):
            pltpu.make_async_copy(table_hbm.at[idx_v.at[k]], acc, gsem).wait()

    def transpose(acc, accT):
        @plsc.parallel_loop(0, HW, 1, unroll=2)
        def _(hw):
            colv = jnp.full((L,), hw, dtype=jnp.int32)
            for c in range(CD):
                plsc.store_scatter(accT, [drow_vecs[c], colv],
                                   acc[hw, pl.ds(c * L, L)])

    bufs = ((acc0, accT0, gsem0, osem0), (acc1, accT1, gsem1, osem1))

    zero_acc(acc0)
    zero_acc(acc1)
    fire_gathers(0, acc0, gsem0)
    fire_gathers(1, acc1, gsem1)

    def pipe(t, c2):
        for p, (acc, accT, gsem, osem) in enumerate(bufs):
            j = t * 2 + p
            wait_gathers(acc, gsem)

            @pl.when(j >= 2)
            def _():
                pltpu.make_async_copy(accT, out_hbm.at[pl.ds(0, D)],
                                      osem).wait()

            transpose(acc, accT)
            zero_acc(acc)

            @pl.when(j + 2 < BPW)
            def _():
                fire_gathers(j + 2, acc, gsem)

            pltpu.async_copy(accT, out_hbm.at[pl.ds((b0 + j) * D, D)], osem)
        return c2

    lax.fori_loop(0, BPW // 2, pipe, 0)
    pltpu.make_async_copy(accT0, out_hbm.at[pl.ds(0, D)], osem0).wait()
    pltpu.make_async_copy(accT1, out_hbm.at[pl.ds(0, D)], osem1).wait()


def kernel(inputs, table):
    inp2 = inputs.reshape(B * K, HW)
    mesh = plsc.VectorSubcoreMesh(
        core_axis_name="c", subcore_axis_name="s",
        num_cores=NC, num_subcores=NS,
    )
    out = pl.kernel(
        _sc_body,
        out_type=jax.ShapeDtypeStruct((B * D, HW), jnp.float32),
        mesh=mesh,
        scratch_types=[
            pltpu.VMEM((BPW * K, HW), jnp.int32),
            pltpu.VMEM((HW, D), jnp.float32),
            pltpu.VMEM((HW, D), jnp.float32),
            pltpu.VMEM((D, HW), jnp.float32),
            pltpu.VMEM((D, HW), jnp.float32),
            pltpu.SemaphoreType.DMA,
            pltpu.SemaphoreType.DMA,
            pltpu.SemaphoreType.DMA,
            pltpu.SemaphoreType.DMA,
        ],
        compiler_params=pltpu.CompilerParams(needs_layout_passes=False),
    )(inp2, table)
    return out.reshape(B, D, H, W)
